# baseline (speedup 1.0000x reference)
import jax
import jax.numpy as jnp
from jax.experimental import pallas as pl

_R = 32
_EPS = 0.0


def _norm_idx_body(coords_ref, norm_ref, idx_ref):
    c = coords_ref[0]
    mean = jnp.mean(c, axis=1, keepdims=True)
    nc = c - mean
    nrm = jnp.sqrt(jnp.sum(nc * nc, axis=0, keepdims=True))
    denom = jnp.max(nrm) * 2.0 + _EPS
    ncn = nc / denom + 0.5
    scaled = jnp.clip(ncn * _R, 0.0, _R - 1.0)
    vox = jnp.round(scaled).astype(jnp.int32)
    norm_ref[0] = scaled
    idx_ref[0] = ((vox[0] * _R + vox[1]) * _R + vox[2])[None, :]


def _norm_idx(coords):
    B, _, N = coords.shape
    norm, idx3 = pl.pallas_call(
        _norm_idx_body,
        grid=(B,),
        in_specs=[pl.BlockSpec((1, 3, N), lambda b: (b, 0, 0))],
        out_specs=[
            pl.BlockSpec((1, 3, N), lambda b: (b, 0, 0)),
            pl.BlockSpec((1, 1, N), lambda b: (b, 0, 0)),
        ],
        out_shape=[
            jax.ShapeDtypeStruct((B, 3, N), jnp.float32),
            jax.ShapeDtypeStruct((B, 1, N), jnp.int32),
        ],
    )(coords)
    return norm, idx3[:, 0, :]


def kernel(features, coords):
    B, C, N = features.shape
    V = _R ** 3
    norm, idx = _norm_idx(coords)

    def one(f, i):
        sums = jax.ops.segment_sum(f.T, i, num_segments=V)
        cnt = jax.ops.segment_sum(jnp.ones(i.shape, dtype=f.dtype), i,
                                  num_segments=V)
        avg = sums / jnp.maximum(cnt, 1.0)[:, None]
        return jnp.transpose(avg).reshape(C, _R, _R, _R)

    out = jax.vmap(one)(features, idx)
    return out, norm


# speedup vs baseline: 1.7602x; 1.7602x over previous
import functools

import jax
import jax.numpy as jnp
from jax import lax
from jax.experimental import pallas as pl
from jax.experimental.pallas import tpu as pltpu
from jax.experimental.pallas import tpu_sc as plsc

_R = 32
_EPS = 0.0
_B, _C, _N = 8, 64, 32768
_V = _R ** 3
_NC, _NS, _L = 2, 16, 16
_NVEC = _N // _L
_FCH = 8192
_UNROLL = 8


def _norm_idx_body(coords_ref, norm_ref, idx_ref):
    c = coords_ref[0]
    mean = jnp.mean(c, axis=1, keepdims=True)
    nc = c - mean
    nrm = jnp.sqrt(jnp.sum(nc * nc, axis=0, keepdims=True))
    denom = jnp.max(nrm) * 2.0 + _EPS
    ncn = nc / denom + 0.5
    scaled = jnp.clip(ncn * _R, 0.0, _R - 1.0)
    vox = jnp.round(scaled).astype(jnp.int32)
    norm_ref[0] = scaled
    idx_ref[0] = ((vox[0] * _R + vox[1]) * _R + vox[2])[None, :]


def _norm_idx(coords):
    B, _, N = coords.shape
    norm, idx3 = pl.pallas_call(
        _norm_idx_body,
        grid=(B,),
        in_specs=[pl.BlockSpec((1, 3, N), lambda b: (b, 0, 0))],
        out_specs=[
            pl.BlockSpec((1, 3, N), lambda b: (b, 0, 0)),
            pl.BlockSpec((1, 1, N), lambda b: (b, 0, 0)),
        ],
        out_shape=[
            jax.ShapeDtypeStruct((B, 3, N), jnp.float32),
            jax.ShapeDtypeStruct((B, 1, N), jnp.int32),
        ],
    )(coords)
    return norm, idx3[:, 0, :]


def _loop(n, body):
    def outer(i, carry):
        for u in range(_UNROLL):
            body(i * _UNROLL + u)
        return carry
    lax.fori_loop(0, n // _UNROLL, outer, 0)


def _sc_body(f_hbm, idx_hbm, out_hbm, idxb, acc, cnt, fchunk):
    cid = lax.axis_index("c")
    sid = lax.axis_index("s")
    wid = sid * _NC + cid
    zero16 = jnp.zeros((_L,), jnp.float32)
    one16 = jnp.ones((_L,), jnp.float32)

    def batch_body(b, carry):
        pltpu.sync_copy(idx_hbm.at[b], idxb)

        def zc(j):
            cnt[pl.ds(j * _L, _L)] = zero16
        _loop(_V // _L, zc)

        def count(j):
            iv = idxb[pl.ds(j * _L, _L)]
            plsc.addupdate_scatter(cnt, [iv], one16)
        _loop(_NVEC, count)

        def recip(j):
            s = pl.ds(j * _L, _L)
            cnt[s] = 1.0 / jnp.maximum(cnt[s], 1.0)
        _loop(_V // _L, recip)

        for half in range(2):
            ch = wid * 2 + half

            def za(j):
                acc[pl.ds(j * _L, _L)] = zero16
            _loop(_V // _L, za)

            for k in range(_N // _FCH):
                pltpu.sync_copy(f_hbm.at[b, ch, pl.ds(k * _FCH, _FCH)],
                                fchunk)

                def accum(j):
                    iv = idxb[pl.ds(k * _FCH + j * _L, _L)]
                    fv = fchunk[pl.ds(j * _L, _L)]
                    plsc.addupdate_scatter(acc, [iv], fv)
                _loop(_FCH // _L, accum)

            def div(j):
                s = pl.ds(j * _L, _L)
                acc[s] = acc[s] * cnt[s]
            _loop(_V // _L, div)

            pltpu.sync_copy(acc, out_hbm.at[b, ch])
        return carry

    lax.fori_loop(0, _B, batch_body, 0)


_sc_scatter = functools.partial(
    pl.kernel,
    out_type=jax.ShapeDtypeStruct((_B, _C, _V), jnp.float32),
    mesh=plsc.VectorSubcoreMesh(core_axis_name="c", subcore_axis_name="s"),
    compiler_params=pltpu.CompilerParams(use_tc_tiling_on_sc=False,
                                         needs_layout_passes=False),
    scratch_types=[
        pltpu.VMEM((_N,), jnp.int32),
        pltpu.VMEM((_V,), jnp.float32),
        pltpu.VMEM((_V,), jnp.float32),
        pltpu.VMEM((_FCH,), jnp.float32),
    ],
)(_sc_body)


def kernel(features, coords):
    norm, idx = _norm_idx(coords)
    out = _sc_scatter(features, idx)
    return out.reshape(_B, _C, _R, _R, _R), norm
